# baseline (speedup 1.0000x reference)
import functools
import math

import jax
import jax.numpy as jnp
from jax import lax
from jax.experimental import pallas as pl
from jax.experimental.pallas import tpu as pltpu
from jax.experimental.pallas import tpu_sc as plsc

_NUM_CORES = 2
_NUM_SUBCORES = 16
_NUM_WORKERS = _NUM_CORES * _NUM_SUBCORES
_CHUNK = 128
_NBUF = 5
_PD = 3


def _make_sc_lookup(B, L, V, D):
  mesh = plsc.VectorSubcoreMesh(core_axis_name="c", subcore_axis_name="s")
  t_per_w = B * L // _NUM_WORKERS
  n_chunks = t_per_w // _CHUNK
  period = L // math.gcd(_CHUNK, L)
  assert t_per_w % _CHUNK == 0 and n_chunks % period == 0
  assert period % _NBUF == 0 and _PD < _NBUF

  @functools.partial(
      pl.kernel,
      out_type=jax.ShapeDtypeStruct((B * L, D), jnp.float32),
      mesh=mesh,
      scratch_types=[
          pltpu.VMEM_SHARED((L, D), jnp.float32),
          pltpu.VMEM((t_per_w,), jnp.int32),
      ] + [pltpu.VMEM((_CHUNK, D), jnp.float32) for _ in range(_NBUF)]
        + [pltpu.SemaphoreType.DMA for _ in range(2 * _NBUF)],
  )
  def lookup(x_hbm, pe_hbm, emb_hbm, out_hbm, pe_sh, idx_v, *bufs_sems):
    rows = bufs_sems[:_NBUF]
    gsem = bufs_sems[_NBUF:2 * _NBUF]
    ssem = bufs_sems[2 * _NBUF:]
    wid = lax.axis_index("s") * _NUM_CORES + lax.axis_index("c")
    base = wid * t_per_w
    pltpu.sync_copy(x_hbm.at[pl.ds(base, t_per_w)], idx_v)

    @pl.when(lax.axis_index("s") == 0)
    def _():
      pltpu.sync_copy(pe_hbm, pe_sh)
    plsc.subcore_barrier()

    def gather_start(c, j):
      p = j % _NBUF
      off = (j * _CHUNK) % L
      sz1 = min(_CHUNK, L - off)
      pltpu.sync_copy(pe_sh.at[pl.ds(off, sz1)], rows[p].at[pl.ds(0, sz1)])
      if sz1 < _CHUNK:
        pltpu.sync_copy(pe_sh.at[pl.ds(0, _CHUNK - sz1)],
                        rows[p].at[pl.ds(sz1, _CHUNK - sz1)])
      pltpu.async_copy(
          emb_hbm.at[idx_v.at[pl.ds(c * _CHUNK, _CHUNK)]],
          rows[p], gsem[p], add=True)

    def gather_drain(p):
      pltpu.make_async_copy(
          emb_hbm.at[pl.ds(0, _CHUNK)], rows[p], gsem[p]).wait()

    def store_start(c, p):
      pltpu.async_copy(
          rows[p], out_hbm.at[pl.ds(base + c * _CHUNK, _CHUNK)], ssem[p])

    def store_drain(p):
      pltpu.make_async_copy(
          emb_hbm.at[pl.ds(0, _CHUNK)], rows[p], ssem[p]).wait()

    for c in range(_PD):
      gather_start(c, c)

    @pl.loop(0, n_chunks, step=period)
    def _(c0):
      for j in range(period):
        c = c0 + j
        p = j % _NBUF
        gather_drain(p)
        store_start(c, p)
        q = (j + _PD) % _NBUF

        @pl.when(c >= _NBUF - _PD)
        def _():
          store_drain(q)

        @pl.when(c + _PD < n_chunks)
        def _():
          gather_start(c + _PD, j + _PD)

    for i in range(_NBUF - _PD):
      store_drain((n_chunks - (_NBUF - _PD) + i) % _NBUF)

  return lookup


def kernel(x, offset, emb, pe):
  B, L = x.shape
  V, D = emb.shape
  pe_s = lax.dynamic_slice_in_dim(pe, offset, L, axis=0)
  out = _make_sc_lookup(B, L, V, D)(x.reshape(-1), pe_s, emb)
  return out.reshape(B, L, D)
